# baseline (speedup 1.0000x reference)
import functools

import jax
import jax.numpy as jnp
from jax.experimental import pallas as pl
from jax.experimental.pallas import tpu as pltpu

N_TOK = 2048
D_IN = 768
D_HID = 2048
D_OUT = 768
RANK = 64
E = 8
NLOW = 2
TOPK = 2

TB = 128
BLK = 256
NB = N_TOK * TOPK // BLK + E
NPB = NB * BLK


def _gate_body(x_ref, wg_ref, bg_ref, e_ref, v_ref):
    logits = (jnp.dot(x_ref[...], wg_ref[...], preferred_element_type=jnp.float32)
              + bg_ref[...])
    lane = jax.lax.broadcasted_iota(jnp.int32, logits.shape, 1)
    m1 = jnp.max(logits, axis=1, keepdims=True)
    i1 = jnp.min(jnp.where(logits == m1, lane, E), axis=1, keepdims=True)
    l2 = jnp.where(lane == i1, -jnp.inf, logits)
    m2 = jnp.max(l2, axis=1, keepdims=True)
    i2 = jnp.min(jnp.where(l2 == m2, lane, E), axis=1, keepdims=True)
    e2 = jnp.exp(m2 - m1)
    s = 1.0 + e2
    e_ref[...] = jnp.concatenate([i1, i2], axis=1)
    v_ref[...] = jnp.concatenate([1.0 / s, e2 / s], axis=1)


def _gate(x, Wg, bg):
    return pl.pallas_call(
        _gate_body,
        grid=(N_TOK // TB,),
        in_specs=[
            pl.BlockSpec((TB, D_IN), lambda t: (t, 0)),
            pl.BlockSpec((D_IN, E), lambda t: (0, 0)),
            pl.BlockSpec((1, E), lambda t: (0, 0)),
        ],
        out_specs=[
            pl.BlockSpec((TB, TOPK), lambda t: (t, 0)),
            pl.BlockSpec((TB, TOPK), lambda t: (t, 0)),
        ],
        out_shape=[
            jax.ShapeDtypeStruct((N_TOK, TOPK), jnp.int32),
            jax.ShapeDtypeStruct((N_TOK, TOPK), jnp.float32),
        ],
    )(x, Wg, bg.reshape(1, E))


def _uv_body(u_ref, v_ref, o_ref):
    o_ref[0] = jnp.dot(u_ref[0], v_ref[0], preferred_element_type=jnp.float32)


def _uv_collapse(U, V):
    return pl.pallas_call(
        _uv_body,
        grid=(NLOW,),
        in_specs=[
            pl.BlockSpec((1, D_HID, RANK), lambda e: (e, 0, 0)),
            pl.BlockSpec((1, RANK, D_OUT), lambda e: (e, 0, 0)),
        ],
        out_specs=pl.BlockSpec((1, D_HID, D_OUT), lambda e: (e, 0, 0)),
        out_shape=jax.ShapeDtypeStruct((NLOW, D_HID, D_OUT), jnp.float32),
    )(U, V)


def _erf(z):
    a = jnp.abs(z)
    t = 1.0 / (1.0 + 0.3275911 * a)
    p = t * (0.254829592 + t * (-0.284496736 + t * (1.421413741
            + t * (-1.453152027 + t * 1.061405429))))
    return jnp.sign(z) * (1.0 - p * jnp.exp(-a * a))


def _gelu_exact(x):
    return 0.5 * x * (1.0 + _erf(x * 0.7071067811865476))


KT = 4
KC = D_HID // KT


def _ffn_body(eb_ref, xs_ref, w1_ref, b1_ref, w2_ref, b2_ref, ws_ref, ys_ref):
    k = pl.program_id(1)
    h = jnp.dot(xs_ref[...], w1_ref[0], preferred_element_type=jnp.float32) + b1_ref[0]
    h = _gelu_exact(h)
    part = jnp.dot(h, w2_ref[0], preferred_element_type=jnp.float32)

    @pl.when(k == 0)
    def _():
        ys_ref[...] = (part + b2_ref[0]) * ws_ref[...]

    @pl.when(k > 0)
    def _():
        ys_ref[...] += part * ws_ref[...]


def _grouped_ffn(eb, xs, W1, b1, W2all, b2all, ws):
    grid_spec = pltpu.PrefetchScalarGridSpec(
        num_scalar_prefetch=1,
        grid=(NB, KT),
        in_specs=[
            pl.BlockSpec((BLK, D_IN), lambda b, k, eb: (b, 0)),
            pl.BlockSpec((1, D_IN, KC), lambda b, k, eb: (eb[b], 0, k)),
            pl.BlockSpec((1, 1, KC), lambda b, k, eb: (eb[b], 0, k)),
            pl.BlockSpec((1, KC, D_OUT), lambda b, k, eb: (eb[b], k, 0)),
            pl.BlockSpec((1, 1, D_OUT), lambda b, k, eb: (eb[b], 0, 0)),
            pl.BlockSpec((BLK, 1), lambda b, k, eb: (b, 0)),
        ],
        out_specs=pl.BlockSpec((BLK, D_OUT), lambda b, k, eb: (b, 0)),
    )
    return pl.pallas_call(
        _ffn_body,
        grid_spec=grid_spec,
        out_shape=jax.ShapeDtypeStruct((NPB, D_OUT), jnp.float32),
    )(eb, xs, W1, b1.reshape(E, 1, D_HID), W2all, b2all.reshape(E, 1, D_OUT),
      ws.reshape(NPB, 1))


def kernel(x, W1, b1, U, V, bl, W2, b2, Wg, bg):
    e_out, v_out = _gate(x, Wg, bg)
    W2all = jnp.concatenate([_uv_collapse(U, V), W2], axis=0)
    b2all = jnp.concatenate([bl, b2], axis=0)

    flat_e = e_out.reshape(-1)
    flat_w = v_out.reshape(-1)
    sort_idx = jnp.argsort(flat_e, stable=True)
    counts = jnp.bincount(flat_e, length=E)
    nb_e = (counts + BLK - 1) // BLK
    cum_incl = jnp.cumsum(nb_e)
    padded_off = (cum_incl - nb_e) * BLK
    g_start = jnp.cumsum(counts) - counts
    k = jnp.arange(N_TOK * TOPK)
    e_k = flat_e[sort_idx]
    row_k = padded_off[e_k] + k - g_start[e_k]
    src = jnp.zeros((NPB,), jnp.int32).at[row_k].set((sort_idx // TOPK).astype(jnp.int32))
    ws = jnp.zeros((NPB,), jnp.float32).at[row_k].set(flat_w[sort_idx])
    dest = jnp.zeros((N_TOK * TOPK,), jnp.int32).at[sort_idx].set(row_k.astype(jnp.int32))
    eb = jnp.minimum(
        jnp.sum(jnp.arange(NB)[:, None] >= cum_incl[None, :], axis=1), E - 1
    ).astype(jnp.int32)
    xs = x[src]

    ys = _grouped_ffn(eb, xs, W1, b1, W2all, b2all, ws)

    d = dest.reshape(N_TOK, TOPK)
    return ys[d[:, 0]] + ys[d[:, 1]]


# speedup vs baseline: 1.5937x; 1.5937x over previous
import functools

import jax
import jax.numpy as jnp
from jax import lax
from jax.experimental import pallas as pl
from jax.experimental.pallas import tpu as pltpu
from jax.experimental.pallas import tpu_sc as plsc

N_TOK = 2048
D_IN = 768
D_HID = 2048
D_OUT = 768
RANK = 64
E = 8
NLOW = 2
TOPK = 2
NFLAT = N_TOK * TOPK

TB = 128
BLK = 256
NB = NFLAT // BLK + E
NPB = NB * BLK

NW = 32
CHUNK = NFLAT // NW
TPW = N_TOK // NW


def _gate_body(x_ref, wg_ref, bg_ref, e_ref, v_ref, r_ref, cnt_ref, carry_ref):
    t = pl.program_id(0)
    logits = (jnp.dot(x_ref[...], wg_ref[...], preferred_element_type=jnp.float32)
              + bg_ref[...])
    lane = lax.broadcasted_iota(jnp.int32, logits.shape, 1)
    m1 = jnp.max(logits, axis=1, keepdims=True)
    i1 = jnp.min(jnp.where(logits == m1, lane, E), axis=1, keepdims=True)
    l2 = jnp.where(lane == i1, -jnp.inf, logits)
    m2 = jnp.max(l2, axis=1, keepdims=True)
    i2 = jnp.min(jnp.where(l2 == m2, lane, E), axis=1, keepdims=True)
    e2 = jnp.exp(m2 - m1)
    s = 1.0 + e2
    e_ref[...] = jnp.concatenate([i1, i2], axis=1)
    v_ref[...] = jnp.concatenate([1.0 / s, e2 / s], axis=1)

    @pl.when(t == 0)
    def _():
        carry_ref[...] = jnp.zeros_like(carry_ref)

    lane16 = lax.broadcasted_iota(jnp.int32, (TB, 16), 1)
    oh1 = (lane16 == i1).astype(jnp.float32)
    oh2 = (lane16 == i2).astype(jnp.float32)
    ohb = oh1 + oh2
    row = lax.broadcasted_iota(jnp.int32, (TB, TB), 0)
    col = lax.broadcasted_iota(jnp.int32, (TB, TB), 1)
    ltri = (row > col).astype(jnp.float32)
    cum = jnp.dot(ltri, ohb, preferred_element_type=jnp.float32) + carry_ref[...]
    r1 = jnp.sum(oh1 * cum, axis=1, keepdims=True)
    r2 = jnp.sum(oh2 * cum, axis=1, keepdims=True)
    r_ref[...] = jnp.concatenate([r1, r2], axis=1).astype(jnp.int32)
    carry_ref[...] += jnp.sum(ohb, axis=0, keepdims=True)
    cnt_ref[...] = carry_ref[...].astype(jnp.int32)


def _gate(x, Wg, bg):
    return pl.pallas_call(
        _gate_body,
        grid=(N_TOK // TB,),
        in_specs=[
            pl.BlockSpec((TB, D_IN), lambda t: (t, 0)),
            pl.BlockSpec((D_IN, E), lambda t: (0, 0)),
            pl.BlockSpec((1, E), lambda t: (0, 0)),
        ],
        out_specs=[
            pl.BlockSpec((TB, TOPK), lambda t: (t, 0)),
            pl.BlockSpec((TB, TOPK), lambda t: (t, 0)),
            pl.BlockSpec((TB, TOPK), lambda t: (t, 0)),
            pl.BlockSpec((1, 16), lambda t: (0, 0)),
        ],
        out_shape=[
            jax.ShapeDtypeStruct((N_TOK, TOPK), jnp.int32),
            jax.ShapeDtypeStruct((N_TOK, TOPK), jnp.float32),
            jax.ShapeDtypeStruct((N_TOK, TOPK), jnp.int32),
            jax.ShapeDtypeStruct((1, 16), jnp.int32),
        ],
        scratch_shapes=[pltpu.VMEM((1, 16), jnp.float32)],
    )(x, Wg, bg.reshape(1, E))


def _fin_body(e_ref, v_ref, r_ref, cnt_ref, dest_ref, vb_ref, eb_ref):
    t = pl.program_id(0)
    cnt = cnt_ref[...]
    nb = (cnt + (BLK - 1)) >> 8
    nbf = nb.astype(jnp.float32)
    erow = lax.broadcasted_iota(jnp.int32, (16, 16), 0)
    ecol = lax.broadcasted_iota(jnp.int32, (16, 16), 1)
    ltri = (erow < ecol).astype(jnp.float32)
    cex = jnp.dot(nbf, ltri, preferred_element_type=jnp.float32)
    po = cex * float(BLK)
    cin = cex + nbf

    i1 = e_ref[:, :1]
    i2 = e_ref[:, 1:2]
    lane16 = lax.broadcasted_iota(jnp.int32, (TB, 16), 1)
    oh1 = (lane16 == i1).astype(jnp.float32)
    oh2 = (lane16 == i2).astype(jnp.float32)
    d1 = jnp.sum(oh1 * po, axis=1, keepdims=True).astype(jnp.int32) + r_ref[:, :1]
    d2 = jnp.sum(oh2 * po, axis=1, keepdims=True).astype(jnp.int32) + r_ref[:, 1:2]
    dest_ref[...] = jnp.concatenate([d1, d2], axis=1)

    z = jnp.zeros((TB, 16), jnp.float32)
    vb_ref[...] = jnp.concatenate([v_ref[:, :1] + z, v_ref[:, 1:2] + z], axis=1)

    @pl.when(t == 0)
    def _():
        bcol = lax.broadcasted_iota(jnp.int32, (NW, 16), 0).astype(jnp.float32)
        used = (bcol >= cin).astype(jnp.int32)
        acc = jnp.sum(used, axis=1, keepdims=True)
        eb_ref[...] = jnp.minimum(acc, E - 1)


def _finalize(e_out, v_out, lrank, cnt):
    return pl.pallas_call(
        _fin_body,
        grid=(N_TOK // TB,),
        in_specs=[
            pl.BlockSpec((TB, TOPK), lambda t: (t, 0)),
            pl.BlockSpec((TB, TOPK), lambda t: (t, 0)),
            pl.BlockSpec((TB, TOPK), lambda t: (t, 0)),
            pl.BlockSpec((1, 16), lambda t: (0, 0)),
        ],
        out_specs=[
            pl.BlockSpec((TB, TOPK), lambda t: (t, 0)),
            pl.BlockSpec((TB, 2 * 16), lambda t: (t, 0)),
            pl.BlockSpec((NW, 1), lambda t: (0, 0)),
        ],
        out_shape=[
            jax.ShapeDtypeStruct((N_TOK, TOPK), jnp.int32),
            jax.ShapeDtypeStruct((N_TOK, 2 * 16), jnp.float32),
            jax.ShapeDtypeStruct((NW, 1), jnp.int32),
        ],
    )(e_out, v_out, lrank, cnt)


def _uv_body(u_ref, v_ref, o_ref):
    o_ref[0] = jnp.dot(u_ref[0], v_ref[0], preferred_element_type=jnp.float32)


def _uv_collapse(U, V):
    return pl.pallas_call(
        _uv_body,
        grid=(NLOW,),
        in_specs=[
            pl.BlockSpec((1, D_HID, RANK), lambda e: (e, 0, 0)),
            pl.BlockSpec((1, RANK, D_OUT), lambda e: (e, 0, 0)),
        ],
        out_specs=pl.BlockSpec((1, D_HID, D_OUT), lambda e: (e, 0, 0)),
        out_shape=jax.ShapeDtypeStruct((NLOW, D_HID, D_OUT), jnp.float32),
    )(U, V)


_SC_MESH = plsc.VectorSubcoreMesh(core_axis_name="c", subcore_axis_name="s")


@functools.partial(
    pl.kernel,
    mesh=_SC_MESH,
    out_type=jax.ShapeDtypeStruct((NPB, D_IN), jnp.float32),
    scratch_types=[
        pltpu.VMEM((CHUNK,), jnp.int32),
        pltpu.VMEM((CHUNK,), jnp.int32),
        pltpu.VMEM((CHUNK, D_IN), jnp.float32),
        pltpu.SemaphoreType.DMA,
    ],
)
def _dispatch(dest_hbm, x_hbm, xs_hbm, destv, tokv, rows, sem):
    w = lax.axis_index("s") * 2 + lax.axis_index("c")
    base = w * CHUNK
    pltpu.sync_copy(dest_hbm.at[pl.ds(base, CHUNK)], destv)
    lane = lax.iota(jnp.int32, 16)
    for j in range(CHUNK // 16):
        tokv[pl.ds(j * 16, 16)] = (base + j * 16 + lane) >> 1
    pltpu.async_copy(x_hbm.at[tokv], rows, sem).wait()
    pltpu.async_copy(rows, xs_hbm.at[destv], sem).wait()


def _erf(z):
    a = jnp.abs(z)
    t = 1.0 / (1.0 + 0.3275911 * a)
    p = t * (0.254829592 + t * (-0.284496736 + t * (1.421413741
            + t * (-1.453152027 + t * 1.061405429))))
    return jnp.sign(z) * (1.0 - p * jnp.exp(-a * a))


def _gelu_exact(x):
    return 0.5 * x * (1.0 + _erf(x * 0.7071067811865476))


def _ffn_body(eb_ref, xs_ref, w1_ref, b1_ref, w2_ref, b2_ref, ys_ref):
    h = jnp.dot(xs_ref[...], w1_ref[0], preferred_element_type=jnp.float32) + b1_ref[0]
    h = _gelu_exact(h)
    ys_ref[...] = jnp.dot(h, w2_ref[0], preferred_element_type=jnp.float32) + b2_ref[0]


def _grouped_ffn(eb, xs, W1, b1, W2all, b2all):
    grid_spec = pltpu.PrefetchScalarGridSpec(
        num_scalar_prefetch=1,
        grid=(NB,),
        in_specs=[
            pl.BlockSpec((BLK, D_IN), lambda b, eb: (b, 0)),
            pl.BlockSpec((1, D_IN, D_HID), lambda b, eb: (eb[b], 0, 0)),
            pl.BlockSpec((1, 1, D_HID), lambda b, eb: (eb[b], 0, 0)),
            pl.BlockSpec((1, D_HID, D_OUT), lambda b, eb: (eb[b], 0, 0)),
            pl.BlockSpec((1, 1, D_OUT), lambda b, eb: (eb[b], 0, 0)),
        ],
        out_specs=pl.BlockSpec((BLK, D_OUT), lambda b, eb: (b, 0)),
    )
    return pl.pallas_call(
        _ffn_body,
        grid_spec=grid_spec,
        out_shape=jax.ShapeDtypeStruct((NPB, D_OUT), jnp.float32),
    )(eb, xs, W1, b1.reshape(E, 1, D_HID), W2all, b2all.reshape(E, 1, D_OUT))


HALF = TPW // 2


@functools.partial(
    pl.kernel,
    mesh=_SC_MESH,
    out_type=jax.ShapeDtypeStruct((N_TOK * D_OUT,), jnp.float32),
    scratch_types=[
        pltpu.VMEM((CHUNK,), jnp.int32),
        pltpu.VMEM((TPW * 32,), jnp.float32),
        pltpu.VMEM((2 * HALF, D_OUT), jnp.float32),
        pltpu.VMEM((HALF * D_OUT,), jnp.float32),
        pltpu.SemaphoreType.DMA,
    ],
)
def _combine(dest_hbm, vb_hbm, ys_hbm, out_hbm, dv, vbv, rows, ob, sem):
    w = lax.axis_index("s") * 2 + lax.axis_index("c")
    base = w * CHUNK
    pltpu.sync_copy(dest_hbm.at[pl.ds(base, CHUNK)], dv)
    pltpu.sync_copy(vb_hbm.at[pl.ds(w * TPW * 32, TPW * 32)], vbv)
    for h in range(2):
        pltpu.async_copy(ys_hbm.at[dv.at[pl.ds(h * 2 * HALF, 2 * HALF)]],
                         rows, sem).wait()

        def body(tl, _):
            tg = h * HALF + tl
            s0 = vbv[pl.ds(tg * 32, 16)]
            s1 = vbv[pl.ds(tg * 32 + 16, 16)]
            for k in range(D_OUT // 16):
                r0 = rows[2 * tl, pl.ds(k * 16, 16)]
                r1 = rows[2 * tl + 1, pl.ds(k * 16, 16)]
                ob[pl.ds(tl * D_OUT + k * 16, 16)] = s0 * r0 + s1 * r1
            return 0

        lax.fori_loop(0, HALF, body, 0)
        pltpu.sync_copy(ob, out_hbm.at[pl.ds((w * TPW + h * HALF) * D_OUT,
                                             HALF * D_OUT)])


def kernel(x, W1, b1, U, V, bl, W2, b2, Wg, bg):
    e_out, v_out, lrank, cnt = _gate(x, Wg, bg)
    dest, vb, eb32 = _finalize(e_out, v_out, lrank, cnt)
    W2all = jnp.concatenate([_uv_collapse(U, V), W2], axis=0)
    b2all = jnp.concatenate([bl, b2], axis=0)

    xs = _dispatch(dest.reshape(NFLAT), x)
    ys = _grouped_ffn(eb32.reshape(NW)[:NB], xs, W1, b1, W2all, b2all)
    out = _combine(dest.reshape(NFLAT), vb.reshape(N_TOK * 32), ys)
    return out.reshape(N_TOK, D_OUT)


# speedup vs baseline: 1.7152x; 1.0763x over previous
import functools

import jax
import jax.numpy as jnp
from jax import lax
from jax.experimental import pallas as pl
from jax.experimental.pallas import tpu as pltpu
from jax.experimental.pallas import tpu_sc as plsc

N_TOK = 2048
D_IN = 768
D_HID = 2048
D_OUT = 768
RANK = 64
E = 8
NLOW = 2
TOPK = 2
NFLAT = N_TOK * TOPK

TB = 128
BLK = 256
NB = NFLAT // BLK + E
NPB = NB * BLK

NW = 32
CHUNK = NFLAT // NW
TPW = N_TOK // NW


def _gate_body(x_ref, wg_ref, bg_ref, e_ref, v_ref, r_ref, cnt_ref, carry_ref):
    t = pl.program_id(0)
    logits = (jnp.dot(x_ref[...], wg_ref[...], preferred_element_type=jnp.float32)
              + bg_ref[...])
    lane = lax.broadcasted_iota(jnp.int32, logits.shape, 1)
    m1 = jnp.max(logits, axis=1, keepdims=True)
    i1 = jnp.min(jnp.where(logits == m1, lane, E), axis=1, keepdims=True)
    l2 = jnp.where(lane == i1, -jnp.inf, logits)
    m2 = jnp.max(l2, axis=1, keepdims=True)
    i2 = jnp.min(jnp.where(l2 == m2, lane, E), axis=1, keepdims=True)
    e2 = jnp.exp(m2 - m1)
    s = 1.0 + e2
    e_ref[...] = jnp.concatenate([i1, i2], axis=1)
    v_ref[...] = jnp.concatenate([1.0 / s, e2 / s], axis=1)

    @pl.when(t == 0)
    def _():
        carry_ref[...] = jnp.zeros_like(carry_ref)

    lane16 = lax.broadcasted_iota(jnp.int32, (TB, 16), 1)
    oh1 = (lane16 == i1).astype(jnp.float32)
    oh2 = (lane16 == i2).astype(jnp.float32)
    ohb = oh1 + oh2
    row = lax.broadcasted_iota(jnp.int32, (TB, TB), 0)
    col = lax.broadcasted_iota(jnp.int32, (TB, TB), 1)
    ltri = (row > col).astype(jnp.float32)
    cum = jnp.dot(ltri, ohb, preferred_element_type=jnp.float32) + carry_ref[...]
    r1 = jnp.sum(oh1 * cum, axis=1, keepdims=True)
    r2 = jnp.sum(oh2 * cum, axis=1, keepdims=True)
    r_ref[...] = jnp.concatenate([r1, r2], axis=1).astype(jnp.int32)
    carry_ref[...] += jnp.sum(ohb, axis=0, keepdims=True)
    cnt_ref[...] = carry_ref[...].astype(jnp.int32)


def _gate(x, Wg, bg):
    return pl.pallas_call(
        _gate_body,
        grid=(N_TOK // TB,),
        in_specs=[
            pl.BlockSpec((TB, D_IN), lambda t: (t, 0)),
            pl.BlockSpec((D_IN, E), lambda t: (0, 0)),
            pl.BlockSpec((1, E), lambda t: (0, 0)),
        ],
        out_specs=[
            pl.BlockSpec((TB, TOPK), lambda t: (t, 0)),
            pl.BlockSpec((TB, TOPK), lambda t: (t, 0)),
            pl.BlockSpec((TB, TOPK), lambda t: (t, 0)),
            pl.BlockSpec((1, 16), lambda t: (0, 0)),
        ],
        out_shape=[
            jax.ShapeDtypeStruct((N_TOK, TOPK), jnp.int32),
            jax.ShapeDtypeStruct((N_TOK, TOPK), jnp.float32),
            jax.ShapeDtypeStruct((N_TOK, TOPK), jnp.int32),
            jax.ShapeDtypeStruct((1, 16), jnp.int32),
        ],
        scratch_shapes=[pltpu.VMEM((1, 16), jnp.float32)],
    )(x, Wg, bg.reshape(1, E))


def _fin_body(e_ref, v_ref, r_ref, cnt_ref, dest_ref, vb_ref, eb_ref):
    t = pl.program_id(0)
    cnt = cnt_ref[...]
    nb = (cnt + (BLK - 1)) >> 8
    nbf = nb.astype(jnp.float32)
    erow = lax.broadcasted_iota(jnp.int32, (16, 16), 0)
    ecol = lax.broadcasted_iota(jnp.int32, (16, 16), 1)
    ltri = (erow < ecol).astype(jnp.float32)
    cex = jnp.dot(nbf, ltri, preferred_element_type=jnp.float32)
    po = cex * float(BLK)
    cin = cex + nbf

    i1 = e_ref[:, :1]
    i2 = e_ref[:, 1:2]
    lane16 = lax.broadcasted_iota(jnp.int32, (TB, 16), 1)
    oh1 = (lane16 == i1).astype(jnp.float32)
    oh2 = (lane16 == i2).astype(jnp.float32)
    d1 = jnp.sum(oh1 * po, axis=1, keepdims=True).astype(jnp.int32) + r_ref[:, :1]
    d2 = jnp.sum(oh2 * po, axis=1, keepdims=True).astype(jnp.int32) + r_ref[:, 1:2]
    dest_ref[...] = jnp.concatenate([d1, d2], axis=1)

    z = jnp.zeros((TB, 16), jnp.float32)
    vb_ref[...] = jnp.concatenate([v_ref[:, :1] + z, v_ref[:, 1:2] + z], axis=1)

    @pl.when(t == 0)
    def _():
        bcol = lax.broadcasted_iota(jnp.int32, (NW, 16), 0).astype(jnp.float32)
        used = (bcol >= cin).astype(jnp.int32)
        acc = jnp.minimum(jnp.sum(used, axis=1, keepdims=True), E - 1)
        brow = lax.broadcasted_iota(jnp.int32, (NW, 1), 0)
        total = cin[:, 7:8].astype(jnp.int32)
        eb_ref[...] = jnp.where(brow == NB, total, acc)


def _finalize(e_out, v_out, lrank, cnt):
    return pl.pallas_call(
        _fin_body,
        grid=(N_TOK // TB,),
        in_specs=[
            pl.BlockSpec((TB, TOPK), lambda t: (t, 0)),
            pl.BlockSpec((TB, TOPK), lambda t: (t, 0)),
            pl.BlockSpec((TB, TOPK), lambda t: (t, 0)),
            pl.BlockSpec((1, 16), lambda t: (0, 0)),
        ],
        out_specs=[
            pl.BlockSpec((TB, TOPK), lambda t: (t, 0)),
            pl.BlockSpec((TB, 2 * 16), lambda t: (t, 0)),
            pl.BlockSpec((NW, 1), lambda t: (0, 0)),
        ],
        out_shape=[
            jax.ShapeDtypeStruct((N_TOK, TOPK), jnp.int32),
            jax.ShapeDtypeStruct((N_TOK, 2 * 16), jnp.float32),
            jax.ShapeDtypeStruct((NW, 1), jnp.int32),
        ],
    )(e_out, v_out, lrank, cnt)


def _uv_body(u_ref, v_ref, o_ref):
    o_ref[0] = jnp.dot(u_ref[0], v_ref[0], preferred_element_type=jnp.float32)


def _uv_collapse(U, V):
    return pl.pallas_call(
        _uv_body,
        grid=(NLOW,),
        in_specs=[
            pl.BlockSpec((1, D_HID, RANK), lambda e: (e, 0, 0)),
            pl.BlockSpec((1, RANK, D_OUT), lambda e: (e, 0, 0)),
        ],
        out_specs=pl.BlockSpec((1, D_HID, D_OUT), lambda e: (e, 0, 0)),
        out_shape=jax.ShapeDtypeStruct((NLOW, D_HID, D_OUT), jnp.float32),
    )(U, V)


_SC_MESH = plsc.VectorSubcoreMesh(core_axis_name="c", subcore_axis_name="s")


@functools.partial(
    pl.kernel,
    mesh=_SC_MESH,
    out_type=jax.ShapeDtypeStruct((NPB, D_IN), jnp.float32),
    scratch_types=[
        pltpu.VMEM((CHUNK,), jnp.int32),
        pltpu.VMEM((CHUNK,), jnp.int32),
        pltpu.VMEM((CHUNK, D_IN), jnp.float32),
        pltpu.SemaphoreType.DMA,
    ],
)
def _dispatch(dest_hbm, x_hbm, xs_hbm, destv, tokv, rows, sem):
    w = lax.axis_index("s") * 2 + lax.axis_index("c")
    base = w * CHUNK
    pltpu.sync_copy(dest_hbm.at[pl.ds(base, CHUNK)], destv)
    lane = lax.iota(jnp.int32, 16)
    for j in range(CHUNK // 16):
        tokv[pl.ds(j * 16, 16)] = (base + j * 16 + lane) >> 1
    pltpu.async_copy(x_hbm.at[tokv], rows, sem).wait()
    pltpu.async_copy(rows, xs_hbm.at[destv], sem).wait()


def _erf(z):
    a = jnp.abs(z)
    t = 1.0 / (1.0 + 0.3275911 * a)
    p = t * (0.254829592 + t * (-0.284496736 + t * (1.421413741
            + t * (-1.453152027 + t * 1.061405429))))
    return jnp.sign(z) * (1.0 - p * jnp.exp(-a * a))


def _gelu_exact(x):
    return 0.5 * x * (1.0 + _erf(x * 0.7071067811865476))


def _ffn_body(eb_ref, xs_ref, w1_ref, b1_ref, w2_ref, b2_ref, ys_ref):
    @pl.when(pl.program_id(0) < eb_ref[NB])
    def _():
        h = (jnp.dot(xs_ref[...], w1_ref[0], preferred_element_type=jnp.float32)
             + b1_ref[0])
        h = _gelu_exact(h)
        ys_ref[...] = (jnp.dot(h, w2_ref[0], preferred_element_type=jnp.float32)
                       + b2_ref[0])


def _grouped_ffn(eb, xs, W1, b1, W2all, b2all):
    grid_spec = pltpu.PrefetchScalarGridSpec(
        num_scalar_prefetch=1,
        grid=(NB,),
        in_specs=[
            pl.BlockSpec((BLK, D_IN), lambda b, eb: (b, 0)),
            pl.BlockSpec((1, D_IN, D_HID), lambda b, eb: (eb[b], 0, 0)),
            pl.BlockSpec((1, 1, D_HID), lambda b, eb: (eb[b], 0, 0)),
            pl.BlockSpec((1, D_HID, D_OUT), lambda b, eb: (eb[b], 0, 0)),
            pl.BlockSpec((1, 1, D_OUT), lambda b, eb: (eb[b], 0, 0)),
        ],
        out_specs=pl.BlockSpec((BLK, D_OUT), lambda b, eb: (b, 0)),
    )
    return pl.pallas_call(
        _ffn_body,
        grid_spec=grid_spec,
        out_shape=jax.ShapeDtypeStruct((NPB, D_OUT), jnp.float32),
    )(eb, xs, W1, b1.reshape(E, 1, D_HID), W2all, b2all.reshape(E, 1, D_OUT))


HALF = TPW // 2


@functools.partial(
    pl.kernel,
    mesh=_SC_MESH,
    out_type=jax.ShapeDtypeStruct((N_TOK * D_OUT,), jnp.float32),
    scratch_types=[
        pltpu.VMEM((CHUNK,), jnp.int32),
        pltpu.VMEM((TPW * 32,), jnp.float32),
        pltpu.VMEM((2 * HALF, D_OUT), jnp.float32),
        pltpu.VMEM((HALF * D_OUT,), jnp.float32),
        pltpu.SemaphoreType.DMA,
    ],
)
def _combine(dest_hbm, vb_hbm, ys_hbm, out_hbm, dv, vbv, rows, ob, sem):
    w = lax.axis_index("s") * 2 + lax.axis_index("c")
    base = w * CHUNK
    pltpu.sync_copy(dest_hbm.at[pl.ds(base, CHUNK)], dv)
    pltpu.sync_copy(vb_hbm.at[pl.ds(w * TPW * 32, TPW * 32)], vbv)
    for h in range(2):
        pltpu.async_copy(ys_hbm.at[dv.at[pl.ds(h * 2 * HALF, 2 * HALF)]],
                         rows, sem).wait()

        def body(tl, _):
            tg = h * HALF + tl
            s0 = vbv[pl.ds(tg * 32, 16)]
            s1 = vbv[pl.ds(tg * 32 + 16, 16)]
            for k in range(D_OUT // 16):
                r0 = rows[2 * tl, pl.ds(k * 16, 16)]
                r1 = rows[2 * tl + 1, pl.ds(k * 16, 16)]
                ob[pl.ds(tl * D_OUT + k * 16, 16)] = s0 * r0 + s1 * r1
            return 0

        lax.fori_loop(0, HALF, body, 0)
        pltpu.sync_copy(ob, out_hbm.at[pl.ds((w * TPW + h * HALF) * D_OUT,
                                             HALF * D_OUT)])


def kernel(x, W1, b1, U, V, bl, W2, b2, Wg, bg):
    e_out, v_out, lrank, cnt = _gate(x, Wg, bg)
    dest, vb, eb32 = _finalize(e_out, v_out, lrank, cnt)
    W2all = jnp.concatenate([_uv_collapse(U, V), W2], axis=0)
    b2all = jnp.concatenate([bl, b2], axis=0)

    xs = _dispatch(dest.reshape(NFLAT), x)
    ys = _grouped_ffn(eb32.reshape(NW)[:NB + 1], xs, W1, b1, W2all, b2all)
    out = _combine(dest.reshape(NFLAT), vb.reshape(N_TOK * 32), ys)
    return out.reshape(N_TOK, D_OUT)


# speedup vs baseline: 1.7773x; 1.0362x over previous
import functools

import jax
import jax.numpy as jnp
from jax import lax
from jax.experimental import pallas as pl
from jax.experimental.pallas import tpu as pltpu
from jax.experimental.pallas import tpu_sc as plsc

N_TOK = 2048
D_IN = 768
D_HID = 2048
D_OUT = 768
RANK = 64
E = 8
NLOW = 2
TOPK = 2
NFLAT = N_TOK * TOPK

TB = 128
BLK = 128
BSH = 7
NB = NFLAT // BLK + E
NPB = NB * BLK
EBR = 48

NW = 32
TPW = N_TOK // NW


def _gate_body(x_ref, wg_ref, bg_ref, e_ref, v_ref, r_ref, cnt_ref, carry_ref):
    t = pl.program_id(0)
    logits = (jnp.dot(x_ref[...], wg_ref[...], preferred_element_type=jnp.float32)
              + bg_ref[...])
    lane = lax.broadcasted_iota(jnp.int32, logits.shape, 1)
    m1 = jnp.max(logits, axis=1, keepdims=True)
    i1 = jnp.min(jnp.where(logits == m1, lane, E), axis=1, keepdims=True)
    l2 = jnp.where(lane == i1, -jnp.inf, logits)
    m2 = jnp.max(l2, axis=1, keepdims=True)
    i2 = jnp.min(jnp.where(l2 == m2, lane, E), axis=1, keepdims=True)
    e2 = jnp.exp(m2 - m1)
    s = 1.0 + e2
    e_ref[...] = jnp.concatenate([i1, i2], axis=1)
    v_ref[...] = jnp.concatenate([1.0 / s, e2 / s], axis=1)

    @pl.when(t == 0)
    def _():
        carry_ref[...] = jnp.zeros_like(carry_ref)

    lane16 = lax.broadcasted_iota(jnp.int32, (TB, 16), 1)
    oh1 = (lane16 == i1).astype(jnp.float32)
    oh2 = (lane16 == i2).astype(jnp.float32)
    ohb = oh1 + oh2
    row = lax.broadcasted_iota(jnp.int32, (TB, TB), 0)
    col = lax.broadcasted_iota(jnp.int32, (TB, TB), 1)
    ltri = (row > col).astype(jnp.float32)
    cum = jnp.dot(ltri, ohb, preferred_element_type=jnp.float32) + carry_ref[...]
    r1 = jnp.sum(oh1 * cum, axis=1, keepdims=True)
    r2 = jnp.sum(oh2 * cum, axis=1, keepdims=True)
    r_ref[...] = jnp.concatenate([r1, r2], axis=1).astype(jnp.int32)
    carry_ref[...] += jnp.sum(ohb, axis=0, keepdims=True)
    cnt_ref[...] = carry_ref[...].astype(jnp.int32)


def _gate(x, Wg, bg):
    return pl.pallas_call(
        _gate_body,
        grid=(N_TOK // TB,),
        in_specs=[
            pl.BlockSpec((TB, D_IN), lambda t: (t, 0)),
            pl.BlockSpec((D_IN, E), lambda t: (0, 0)),
            pl.BlockSpec((1, E), lambda t: (0, 0)),
        ],
        out_specs=[
            pl.BlockSpec((TB, TOPK), lambda t: (t, 0)),
            pl.BlockSpec((TB, TOPK), lambda t: (t, 0)),
            pl.BlockSpec((TB, TOPK), lambda t: (t, 0)),
            pl.BlockSpec((1, 16), lambda t: (0, 0)),
        ],
        out_shape=[
            jax.ShapeDtypeStruct((N_TOK, TOPK), jnp.int32),
            jax.ShapeDtypeStruct((N_TOK, TOPK), jnp.float32),
            jax.ShapeDtypeStruct((N_TOK, TOPK), jnp.int32),
            jax.ShapeDtypeStruct((1, 16), jnp.int32),
        ],
        scratch_shapes=[pltpu.VMEM((1, 16), jnp.float32)],
    )(x, Wg, bg.reshape(1, E))


def _fin_body(e_ref, v_ref, r_ref, cnt_ref,
              d1_ref, d2_ref, vb1_ref, vb2_ref, eb_ref):
    t = pl.program_id(0)
    cnt = cnt_ref[...]
    nb = (cnt + (BLK - 1)) >> BSH
    nbf = nb.astype(jnp.float32)
    erow = lax.broadcasted_iota(jnp.int32, (16, 16), 0)
    ecol = lax.broadcasted_iota(jnp.int32, (16, 16), 1)
    ltri = (erow < ecol).astype(jnp.float32)
    cex = jnp.dot(nbf, ltri, preferred_element_type=jnp.float32)
    po = cex * float(BLK)
    cin = cex + nbf

    i1 = e_ref[:, :1]
    i2 = e_ref[:, 1:2]
    lane16 = lax.broadcasted_iota(jnp.int32, (TB, 16), 1)
    oh1 = (lane16 == i1).astype(jnp.float32)
    oh2 = (lane16 == i2).astype(jnp.float32)
    d1_ref[...] = (jnp.sum(oh1 * po, axis=1, keepdims=True).astype(jnp.int32)
                   + r_ref[:, :1])
    d2_ref[...] = (jnp.sum(oh2 * po, axis=1, keepdims=True).astype(jnp.int32)
                   + r_ref[:, 1:2])

    z = jnp.zeros((TB, 128), jnp.float32)
    vb1_ref[...] = v_ref[:, :1] + z
    vb2_ref[...] = v_ref[:, 1:2] + z

    @pl.when(t == 0)
    def _():
        bcol = lax.broadcasted_iota(jnp.int32, (EBR, 16), 0).astype(jnp.float32)
        used = (bcol >= cin).astype(jnp.int32)
        acc = jnp.minimum(jnp.sum(used, axis=1, keepdims=True), E - 1)
        brow = lax.broadcasted_iota(jnp.int32, (EBR, 1), 0)
        total = cin[:, 7:8].astype(jnp.int32)
        eb_ref[...] = jnp.where(brow == NB, total, acc)


def _finalize(e_out, v_out, lrank, cnt):
    return pl.pallas_call(
        _fin_body,
        grid=(N_TOK // TB,),
        in_specs=[
            pl.BlockSpec((TB, TOPK), lambda t: (t, 0)),
            pl.BlockSpec((TB, TOPK), lambda t: (t, 0)),
            pl.BlockSpec((TB, TOPK), lambda t: (t, 0)),
            pl.BlockSpec((1, 16), lambda t: (0, 0)),
        ],
        out_specs=[
            pl.BlockSpec((TB, 1), lambda t: (t, 0)),
            pl.BlockSpec((TB, 1), lambda t: (t, 0)),
            pl.BlockSpec((TB, 128), lambda t: (t, 0)),
            pl.BlockSpec((TB, 128), lambda t: (t, 0)),
            pl.BlockSpec((EBR, 1), lambda t: (0, 0)),
        ],
        out_shape=[
            jax.ShapeDtypeStruct((N_TOK, 1), jnp.int32),
            jax.ShapeDtypeStruct((N_TOK, 1), jnp.int32),
            jax.ShapeDtypeStruct((N_TOK, 128), jnp.float32),
            jax.ShapeDtypeStruct((N_TOK, 128), jnp.float32),
            jax.ShapeDtypeStruct((EBR, 1), jnp.int32),
        ],
    )(e_out, v_out, lrank, cnt)


def _uv_body(u_ref, v_ref, o_ref):
    o_ref[0] = jnp.dot(u_ref[0], v_ref[0], preferred_element_type=jnp.float32)


def _uv_collapse(U, V):
    return pl.pallas_call(
        _uv_body,
        grid=(NLOW,),
        in_specs=[
            pl.BlockSpec((1, D_HID, RANK), lambda e: (e, 0, 0)),
            pl.BlockSpec((1, RANK, D_OUT), lambda e: (e, 0, 0)),
        ],
        out_specs=pl.BlockSpec((1, D_HID, D_OUT), lambda e: (e, 0, 0)),
        out_shape=jax.ShapeDtypeStruct((NLOW, D_HID, D_OUT), jnp.float32),
    )(U, V)


_SC_MESH = plsc.VectorSubcoreMesh(core_axis_name="c", subcore_axis_name="s")


@functools.partial(
    pl.kernel,
    mesh=_SC_MESH,
    out_type=[
        jax.ShapeDtypeStruct((NPB, D_IN), jnp.float32),
        jax.ShapeDtypeStruct((NPB, 128), jnp.float32),
    ],
    scratch_types=[
        pltpu.VMEM((TPW,), jnp.int32),
        pltpu.VMEM((TPW,), jnp.int32),
        pltpu.VMEM((TPW,), jnp.int32),
        pltpu.VMEM((TPW, D_IN), jnp.float32),
        pltpu.VMEM((TPW, 128), jnp.float32),
        pltpu.VMEM((TPW, 128), jnp.float32),
        pltpu.SemaphoreType.DMA,
        pltpu.SemaphoreType.DMA,
    ],
)
def _dispatch(d1_hbm, d2_hbm, vb1_hbm, vb2_hbm, x_hbm, xs_hbm, xsw_hbm,
              d1v, d2v, tokv, rows, w1v, w2v, sem0, sem1):
    w = lax.axis_index("s") * 2 + lax.axis_index("c")
    base = w * TPW
    pltpu.sync_copy(d1_hbm.at[pl.ds(base, TPW)], d1v)
    pltpu.sync_copy(d2_hbm.at[pl.ds(base, TPW)], d2v)
    pltpu.sync_copy(vb1_hbm.at[pl.ds(base, TPW)], w1v)
    pltpu.sync_copy(vb2_hbm.at[pl.ds(base, TPW)], w2v)
    lane = lax.iota(jnp.int32, 16)
    for j in range(TPW // 16):
        tokv[pl.ds(j * 16, 16)] = base + j * 16 + lane
    pltpu.async_copy(x_hbm.at[tokv], rows, sem0).wait()
    c0 = pltpu.async_copy(rows, xs_hbm.at[d1v], sem0)
    c1 = pltpu.async_copy(rows, xs_hbm.at[d2v], sem1)
    c0.wait()
    c1.wait()
    c2 = pltpu.async_copy(w1v, xsw_hbm.at[d1v], sem0)
    c3 = pltpu.async_copy(w2v, xsw_hbm.at[d2v], sem1)
    c2.wait()
    c3.wait()


def _erf(z):
    a = jnp.abs(z)
    t = 1.0 / (1.0 + 0.3275911 * a)
    p = t * (0.254829592 + t * (-0.284496736 + t * (1.421413741
            + t * (-1.453152027 + t * 1.061405429))))
    return jnp.sign(z) * (1.0 - p * jnp.exp(-a * a))


def _gelu_exact(x):
    return 0.5 * x * (1.0 + _erf(x * 0.7071067811865476))


def _ffn_body(eb_ref, xs_ref, w1_ref, b1_ref, w2_ref, b2_ref, ws_ref, ys_ref):
    @pl.when(pl.program_id(0) < eb_ref[NB])
    def _():
        h = (jnp.dot(xs_ref[...], w1_ref[0], preferred_element_type=jnp.float32)
             + b1_ref[0])
        h = _gelu_exact(h)
        y = (jnp.dot(h, w2_ref[0], preferred_element_type=jnp.float32)
             + b2_ref[0])
        ys_ref[...] = y * ws_ref[:, :1]


def _grouped_ffn(eb, xs, W1, b1, W2all, b2all, xsw):
    grid_spec = pltpu.PrefetchScalarGridSpec(
        num_scalar_prefetch=1,
        grid=(NB,),
        in_specs=[
            pl.BlockSpec((BLK, D_IN), lambda b, eb: (b, 0)),
            pl.BlockSpec((1, D_IN, D_HID), lambda b, eb: (eb[b], 0, 0)),
            pl.BlockSpec((1, 1, D_HID), lambda b, eb: (eb[b], 0, 0)),
            pl.BlockSpec((1, D_HID, D_OUT), lambda b, eb: (eb[b], 0, 0)),
            pl.BlockSpec((1, 1, D_OUT), lambda b, eb: (eb[b], 0, 0)),
            pl.BlockSpec((BLK, 128), lambda b, eb: (b, 0)),
        ],
        out_specs=pl.BlockSpec((BLK, D_OUT), lambda b, eb: (b, 0)),
    )
    return pl.pallas_call(
        _ffn_body,
        grid_spec=grid_spec,
        out_shape=jax.ShapeDtypeStruct((NPB, D_OUT), jnp.float32),
    )(eb, xs, W1, b1.reshape(E, 1, D_HID), W2all, b2all.reshape(E, 1, D_OUT), xsw)


@functools.partial(
    pl.kernel,
    mesh=_SC_MESH,
    out_type=jax.ShapeDtypeStruct((N_TOK, D_OUT), jnp.float32),
    scratch_types=[
        pltpu.VMEM((TPW,), jnp.int32),
        pltpu.VMEM((TPW,), jnp.int32),
        pltpu.VMEM((TPW, D_OUT), jnp.float32),
        pltpu.VMEM((TPW, D_OUT), jnp.float32),
        pltpu.SemaphoreType.DMA,
        pltpu.SemaphoreType.DMA,
    ],
)
def _combine(d1_hbm, d2_hbm, ys_hbm, out_hbm, d1v, d2v, b0, b1, sem0, sem1):
    w = lax.axis_index("s") * 2 + lax.axis_index("c")
    base = w * TPW
    pltpu.sync_copy(d1_hbm.at[pl.ds(base, TPW)], d1v)
    pltpu.sync_copy(d2_hbm.at[pl.ds(base, TPW)], d2v)
    c0 = pltpu.async_copy(ys_hbm.at[d1v], b0, sem0)
    c1 = pltpu.async_copy(ys_hbm.at[d2v], b1, sem1)
    c0.wait()
    c1.wait()

    def body(tl, _):
        for k in range(D_OUT // 16):
            b0[tl, pl.ds(k * 16, 16)] += b1[tl, pl.ds(k * 16, 16)]
        return 0

    lax.fori_loop(0, TPW, body, 0)
    pltpu.sync_copy(b0, out_hbm.at[pl.ds(base, TPW)])


def kernel(x, W1, b1, U, V, bl, W2, b2, Wg, bg):
    e_out, v_out, lrank, cnt = _gate(x, Wg, bg)
    d1, d2, vb1, vb2, eb = _finalize(e_out, v_out, lrank, cnt)
    W2all = jnp.concatenate([_uv_collapse(U, V), W2], axis=0)
    b2all = jnp.concatenate([bl, b2], axis=0)

    d1f = d1.reshape(N_TOK)
    d2f = d2.reshape(N_TOK)
    xs, xsw = _dispatch(d1f, d2f, vb1, vb2, x)
    ys = _grouped_ffn(eb.reshape(EBR)[:NB + 1], xs, W1, b1, W2all, b2all, xsw)
    return _combine(d1f, d2f, ys)


# speedup vs baseline: 1.9186x; 1.0795x over previous
import functools

import jax
import jax.numpy as jnp
from jax import lax
from jax.experimental import pallas as pl
from jax.experimental.pallas import tpu as pltpu
from jax.experimental.pallas import tpu_sc as plsc

N_TOK = 2048
D_IN = 768
D_HID = 2048
D_OUT = 768
RANK = 64
E = 8
NLOW = 2
TOPK = 2
NFLAT = N_TOK * TOPK

TB = 128
BLK = 256
BSH = 8
NB = NFLAT // BLK + E
NPB = NB * BLK
EBR = 32

NW = 32
TPW = N_TOK // NW


def _gate_body(x_ref, wg_ref, bg_ref, e_ref, v_ref, r_ref, cnt_ref, carry_ref):
    t = pl.program_id(0)
    logits = (jnp.dot(x_ref[...], wg_ref[...], preferred_element_type=jnp.float32)
              + bg_ref[...])
    lane = lax.broadcasted_iota(jnp.int32, logits.shape, 1)
    m1 = jnp.max(logits, axis=1, keepdims=True)
    i1 = jnp.min(jnp.where(logits == m1, lane, E), axis=1, keepdims=True)
    l2 = jnp.where(lane == i1, -jnp.inf, logits)
    m2 = jnp.max(l2, axis=1, keepdims=True)
    i2 = jnp.min(jnp.where(l2 == m2, lane, E), axis=1, keepdims=True)
    e2 = jnp.exp(m2 - m1)
    s = 1.0 + e2
    e_ref[...] = jnp.concatenate([i1, i2], axis=1)
    v_ref[...] = jnp.concatenate([1.0 / s, e2 / s], axis=1)

    @pl.when(t == 0)
    def _():
        carry_ref[...] = jnp.zeros_like(carry_ref)

    lane16 = lax.broadcasted_iota(jnp.int32, (TB, 16), 1)
    oh1 = (lane16 == i1).astype(jnp.float32)
    oh2 = (lane16 == i2).astype(jnp.float32)
    ohb = oh1 + oh2
    row = lax.broadcasted_iota(jnp.int32, (TB, TB), 0)
    col = lax.broadcasted_iota(jnp.int32, (TB, TB), 1)
    ltri = (row > col).astype(jnp.float32)
    cum = jnp.dot(ltri, ohb, preferred_element_type=jnp.float32) + carry_ref[...]
    r1 = jnp.sum(oh1 * cum, axis=1, keepdims=True)
    r2 = jnp.sum(oh2 * cum, axis=1, keepdims=True)
    r_ref[...] = jnp.concatenate([r1, r2], axis=1).astype(jnp.int32)
    carry_ref[...] += jnp.sum(ohb, axis=0, keepdims=True)
    cnt_ref[...] = carry_ref[...].astype(jnp.int32)


def _gate(x, Wg, bg):
    return pl.pallas_call(
        _gate_body,
        grid=(N_TOK // TB,),
        in_specs=[
            pl.BlockSpec((TB, D_IN), lambda t: (t, 0)),
            pl.BlockSpec((D_IN, E), lambda t: (0, 0)),
            pl.BlockSpec((1, E), lambda t: (0, 0)),
        ],
        out_specs=[
            pl.BlockSpec((TB, TOPK), lambda t: (t, 0)),
            pl.BlockSpec((TB, TOPK), lambda t: (t, 0)),
            pl.BlockSpec((TB, TOPK), lambda t: (t, 0)),
            pl.BlockSpec((1, 16), lambda t: (0, 0)),
        ],
        out_shape=[
            jax.ShapeDtypeStruct((N_TOK, TOPK), jnp.int32),
            jax.ShapeDtypeStruct((N_TOK, TOPK), jnp.float32),
            jax.ShapeDtypeStruct((N_TOK, TOPK), jnp.int32),
            jax.ShapeDtypeStruct((1, 16), jnp.int32),
        ],
        scratch_shapes=[pltpu.VMEM((1, 16), jnp.float32)],
    )(x, Wg, bg.reshape(1, E))


def _fin_body(e_ref, v_ref, r_ref, cnt_ref,
              d1_ref, d2_ref, vb1_ref, vb2_ref, eb_ref):
    t = pl.program_id(0)
    cnt = cnt_ref[...]
    nb = (cnt + (BLK - 1)) >> BSH
    nbf = nb.astype(jnp.float32)
    erow = lax.broadcasted_iota(jnp.int32, (16, 16), 0)
    ecol = lax.broadcasted_iota(jnp.int32, (16, 16), 1)
    ltri = (erow < ecol).astype(jnp.float32)
    cex = jnp.dot(nbf, ltri, preferred_element_type=jnp.float32)
    po = cex * float(BLK)
    cin = cex + nbf

    i1 = e_ref[:, :1]
    i2 = e_ref[:, 1:2]
    lane16 = lax.broadcasted_iota(jnp.int32, (TB, 16), 1)
    oh1 = (lane16 == i1).astype(jnp.float32)
    oh2 = (lane16 == i2).astype(jnp.float32)
    d1_ref[...] = (jnp.sum(oh1 * po, axis=1, keepdims=True).astype(jnp.int32)
                   + r_ref[:, :1])
    d2_ref[...] = (jnp.sum(oh2 * po, axis=1, keepdims=True).astype(jnp.int32)
                   + r_ref[:, 1:2])

    z = jnp.zeros((TB, 128), jnp.float32)
    vb1_ref[...] = v_ref[:, :1] + z
    vb2_ref[...] = v_ref[:, 1:2] + z

    @pl.when(t == 0)
    def _():
        bcol = lax.broadcasted_iota(jnp.int32, (EBR, 16), 0).astype(jnp.float32)
        used = (bcol >= cin).astype(jnp.int32)
        acc = jnp.minimum(jnp.sum(used, axis=1, keepdims=True), E - 1)
        brow = lax.broadcasted_iota(jnp.int32, (EBR, 1), 0)
        total = cin[:, 7:8].astype(jnp.int32)
        eb_ref[...] = jnp.where(brow == NB, total, acc)


def _finalize(e_out, v_out, lrank, cnt):
    return pl.pallas_call(
        _fin_body,
        grid=(N_TOK // TB,),
        in_specs=[
            pl.BlockSpec((TB, TOPK), lambda t: (t, 0)),
            pl.BlockSpec((TB, TOPK), lambda t: (t, 0)),
            pl.BlockSpec((TB, TOPK), lambda t: (t, 0)),
            pl.BlockSpec((1, 16), lambda t: (0, 0)),
        ],
        out_specs=[
            pl.BlockSpec((TB, 1), lambda t: (t, 0)),
            pl.BlockSpec((TB, 1), lambda t: (t, 0)),
            pl.BlockSpec((TB, 128), lambda t: (t, 0)),
            pl.BlockSpec((TB, 128), lambda t: (t, 0)),
            pl.BlockSpec((EBR, 1), lambda t: (0, 0)),
        ],
        out_shape=[
            jax.ShapeDtypeStruct((N_TOK, 1), jnp.int32),
            jax.ShapeDtypeStruct((N_TOK, 1), jnp.int32),
            jax.ShapeDtypeStruct((N_TOK, 128), jnp.float32),
            jax.ShapeDtypeStruct((N_TOK, 128), jnp.float32),
            jax.ShapeDtypeStruct((EBR, 1), jnp.int32),
        ],
    )(e_out, v_out, lrank, cnt)


def _uv_body(u_ref, v_ref, o_ref):
    o_ref[0] = jnp.dot(u_ref[0], v_ref[0], preferred_element_type=jnp.float32)


def _uv_collapse(U, V):
    return pl.pallas_call(
        _uv_body,
        grid=(NLOW,),
        in_specs=[
            pl.BlockSpec((1, D_HID, RANK), lambda e: (e, 0, 0)),
            pl.BlockSpec((1, RANK, D_OUT), lambda e: (e, 0, 0)),
        ],
        out_specs=pl.BlockSpec((1, D_HID, D_OUT), lambda e: (e, 0, 0)),
        out_shape=jax.ShapeDtypeStruct((NLOW, D_HID, D_OUT), jnp.float32),
    )(U, V)


_SC_MESH = plsc.VectorSubcoreMesh(core_axis_name="c", subcore_axis_name="s")


@functools.partial(
    pl.kernel,
    mesh=_SC_MESH,
    out_type=[
        jax.ShapeDtypeStruct((NPB, D_IN), jnp.float32),
        jax.ShapeDtypeStruct((NPB, 128), jnp.float32),
    ],
    scratch_types=[
        pltpu.VMEM((TPW,), jnp.int32),
        pltpu.VMEM((TPW,), jnp.int32),
        pltpu.VMEM((TPW,), jnp.int32),
        pltpu.VMEM((TPW, D_IN), jnp.float32),
        pltpu.VMEM((TPW, 128), jnp.float32),
        pltpu.VMEM((TPW, 128), jnp.float32),
        pltpu.SemaphoreType.DMA,
        pltpu.SemaphoreType.DMA,
    ],
)
def _dispatch(d1_hbm, d2_hbm, vb1_hbm, vb2_hbm, x_hbm, xs_hbm, xsw_hbm,
              d1v, d2v, tokv, rows, w1v, w2v, sem0, sem1):
    w = lax.axis_index("s") * 2 + lax.axis_index("c")
    base = w * TPW
    pltpu.sync_copy(d1_hbm.at[pl.ds(base, TPW)], d1v)
    pltpu.sync_copy(d2_hbm.at[pl.ds(base, TPW)], d2v)
    pltpu.sync_copy(vb1_hbm.at[pl.ds(base, TPW)], w1v)
    pltpu.sync_copy(vb2_hbm.at[pl.ds(base, TPW)], w2v)
    lane = lax.iota(jnp.int32, 16)
    for j in range(TPW // 16):
        tokv[pl.ds(j * 16, 16)] = base + j * 16 + lane
    pltpu.async_copy(x_hbm.at[tokv], rows, sem0).wait()
    c0 = pltpu.async_copy(rows, xs_hbm.at[d1v], sem0)
    c1 = pltpu.async_copy(rows, xs_hbm.at[d2v], sem1)
    c0.wait()
    c1.wait()
    c2 = pltpu.async_copy(w1v, xsw_hbm.at[d1v], sem0)
    c3 = pltpu.async_copy(w2v, xsw_hbm.at[d2v], sem1)
    c2.wait()
    c3.wait()


def _erf(z):
    a = jnp.abs(z)
    t = 1.0 / (1.0 + 0.3275911 * a)
    p = t * (0.254829592 + t * (-0.284496736 + t * (1.421413741
            + t * (-1.453152027 + t * 1.061405429))))
    return jnp.sign(z) * (1.0 - p * jnp.exp(-a * a))


def _gelu_exact(x):
    return 0.5 * x * (1.0 + _erf(x * 0.7071067811865476))


def _ffn_body(eb_ref, xs_ref, w1_ref, b1_ref, w2_ref, b2_ref, ws_ref, ys_ref):
    @pl.when(pl.program_id(0) < eb_ref[NB])
    def _():
        h = (jnp.dot(xs_ref[...], w1_ref[0], preferred_element_type=jnp.float32)
             + b1_ref[0])
        h = _gelu_exact(h)
        y = (jnp.dot(h, w2_ref[0], preferred_element_type=jnp.float32)
             + b2_ref[0])
        ys_ref[...] = y * ws_ref[:, :1]


def _grouped_ffn(eb, xs, W1, b1, W2all, b2all, xsw):
    grid_spec = pltpu.PrefetchScalarGridSpec(
        num_scalar_prefetch=1,
        grid=(NB,),
        in_specs=[
            pl.BlockSpec((BLK, D_IN), lambda b, eb: (b, 0)),
            pl.BlockSpec((1, D_IN, D_HID), lambda b, eb: (eb[b], 0, 0)),
            pl.BlockSpec((1, 1, D_HID), lambda b, eb: (eb[b], 0, 0)),
            pl.BlockSpec((1, D_HID, D_OUT), lambda b, eb: (eb[b], 0, 0)),
            pl.BlockSpec((1, 1, D_OUT), lambda b, eb: (eb[b], 0, 0)),
            pl.BlockSpec((BLK, 128), lambda b, eb: (b, 0)),
        ],
        out_specs=pl.BlockSpec((BLK, D_OUT), lambda b, eb: (b, 0)),
    )
    return pl.pallas_call(
        _ffn_body,
        grid_spec=grid_spec,
        out_shape=jax.ShapeDtypeStruct((NPB, D_OUT), jnp.float32),
    )(eb, xs, W1, b1.reshape(E, 1, D_HID), W2all, b2all.reshape(E, 1, D_OUT), xsw)


@functools.partial(
    pl.kernel,
    mesh=_SC_MESH,
    out_type=jax.ShapeDtypeStruct((N_TOK, D_OUT), jnp.float32),
    scratch_types=[
        pltpu.VMEM((TPW,), jnp.int32),
        pltpu.VMEM((TPW,), jnp.int32),
        pltpu.VMEM((TPW, D_OUT), jnp.float32),
        pltpu.VMEM((TPW, D_OUT), jnp.float32),
        pltpu.SemaphoreType.DMA,
        pltpu.SemaphoreType.DMA,
    ],
)
def _combine(d1_hbm, d2_hbm, ys_hbm, out_hbm, d1v, d2v, b0, b1, sem0, sem1):
    w = lax.axis_index("s") * 2 + lax.axis_index("c")
    base = w * TPW
    pltpu.sync_copy(d1_hbm.at[pl.ds(base, TPW)], d1v)
    pltpu.sync_copy(d2_hbm.at[pl.ds(base, TPW)], d2v)
    c0 = pltpu.async_copy(ys_hbm.at[d1v], b0, sem0)
    c1 = pltpu.async_copy(ys_hbm.at[d2v], b1, sem1)
    c0.wait()
    c1.wait()

    def body(tl, _):
        for k in range(D_OUT // 16):
            b0[tl, pl.ds(k * 16, 16)] += b1[tl, pl.ds(k * 16, 16)]
        return 0

    lax.fori_loop(0, TPW, body, 0)
    pltpu.sync_copy(b0, out_hbm.at[pl.ds(base, TPW)])


def kernel(x, W1, b1, U, V, bl, W2, b2, Wg, bg):
    e_out, v_out, lrank, cnt = _gate(x, Wg, bg)
    d1, d2, vb1, vb2, eb = _finalize(e_out, v_out, lrank, cnt)
    W2all = jnp.concatenate([_uv_collapse(U, V), W2], axis=0)
    b2all = jnp.concatenate([bl, b2], axis=0)

    d1f = d1.reshape(N_TOK)
    d2f = d2.reshape(N_TOK)
    xs, xsw = _dispatch(d1f, d2f, vb1, vb2, x)
    ys = _grouped_ffn(eb.reshape(EBR)[:NB + 1], xs, W1, b1, W2all, b2all, xsw)
    return _combine(d1f, d2f, ys)


# speedup vs baseline: 1.9466x; 1.0146x over previous
import functools

import jax
import jax.numpy as jnp
from jax import lax
from jax.experimental import pallas as pl
from jax.experimental.pallas import tpu as pltpu
from jax.experimental.pallas import tpu_sc as plsc

N_TOK = 2048
D_IN = 768
D_HID = 2048
D_OUT = 768
RANK = 64
E = 8
NLOW = 2
TOPK = 2
NFLAT = N_TOK * TOPK

TB = 128
BLK = 256
BSH = 8
NB = NFLAT // BLK + E
NPB = NB * BLK
EBR = 32

NW = 32
TPW = N_TOK // NW


def _gate_body(x_ref, wg_ref, bg_ref,
               d1_ref, d2_ref, vb1_ref, vb2_ref, eb_ref,
               ev_s, vv_s, r_s, carry_ref):
    p = pl.program_id(0)
    t = pl.program_id(1)
    lane16 = lax.broadcasted_iota(jnp.int32, (TB, 16), 1)

    @pl.when(p == 0)
    def _gate_pass():
        logits = (jnp.dot(x_ref[...], wg_ref[...],
                          preferred_element_type=jnp.float32)
                  + bg_ref[...])
        lane = lax.broadcasted_iota(jnp.int32, logits.shape, 1)
        m1 = jnp.max(logits, axis=1, keepdims=True)
        i1 = jnp.min(jnp.where(logits == m1, lane, E), axis=1, keepdims=True)
        l2 = jnp.where(lane == i1, -jnp.inf, logits)
        m2 = jnp.max(l2, axis=1, keepdims=True)
        i2 = jnp.min(jnp.where(l2 == m2, lane, E), axis=1, keepdims=True)
        e2 = jnp.exp(m2 - m1)
        s = 1.0 + e2
        ev_s[pl.ds(t * TB, TB), :] = jnp.concatenate([i1, i2], axis=1)
        vv_s[pl.ds(t * TB, TB), :] = jnp.concatenate([1.0 / s, e2 / s], axis=1)

        @pl.when(t == 0)
        def _():
            carry_ref[...] = jnp.zeros_like(carry_ref)

        oh1 = (lane16 == i1).astype(jnp.float32)
        oh2 = (lane16 == i2).astype(jnp.float32)
        ohb = oh1 + oh2
        row = lax.broadcasted_iota(jnp.int32, (TB, TB), 0)
        col = lax.broadcasted_iota(jnp.int32, (TB, TB), 1)
        ltri = (row > col).astype(jnp.float32)
        cum = (jnp.dot(ltri, ohb, preferred_element_type=jnp.float32)
               + carry_ref[...])
        r1 = jnp.sum(oh1 * cum, axis=1, keepdims=True)
        r2 = jnp.sum(oh2 * cum, axis=1, keepdims=True)
        r_s[pl.ds(t * TB, TB), :] = (
            jnp.concatenate([r1, r2], axis=1).astype(jnp.int32))
        carry_ref[...] += jnp.sum(ohb, axis=0, keepdims=True)

    @pl.when(p == 1)
    def _finalize_pass():
        cnt = carry_ref[...].astype(jnp.int32)
        nb = (cnt + (BLK - 1)) >> BSH
        nbf = nb.astype(jnp.float32)
        erow = lax.broadcasted_iota(jnp.int32, (16, 16), 0)
        ecol = lax.broadcasted_iota(jnp.int32, (16, 16), 1)
        ltri = (erow < ecol).astype(jnp.float32)
        cex = jnp.dot(nbf, ltri, preferred_element_type=jnp.float32)
        po = cex * float(BLK)
        cin = cex + nbf

        ev = ev_s[pl.ds(t * TB, TB), :]
        rr = r_s[pl.ds(t * TB, TB), :]
        vv = vv_s[pl.ds(t * TB, TB), :]
        i1 = ev[:, :1]
        i2 = ev[:, 1:2]
        oh1 = (lane16 == i1).astype(jnp.float32)
        oh2 = (lane16 == i2).astype(jnp.float32)
        d1_ref[...] = (jnp.sum(oh1 * po, axis=1, keepdims=True).astype(jnp.int32)
                       + rr[:, :1])
        d2_ref[...] = (jnp.sum(oh2 * po, axis=1, keepdims=True).astype(jnp.int32)
                       + rr[:, 1:2])

        z = jnp.zeros((TB, 128), jnp.float32)
        vb1_ref[...] = vv[:, :1] + z
        vb2_ref[...] = vv[:, 1:2] + z

        @pl.when(t == 0)
        def _():
            bcol = lax.broadcasted_iota(jnp.int32, (EBR, 16), 0).astype(jnp.float32)
            used = (bcol >= cin).astype(jnp.int32)
            acc = jnp.minimum(jnp.sum(used, axis=1, keepdims=True), E - 1)
            brow = lax.broadcasted_iota(jnp.int32, (EBR, 1), 0)
            total = cin[:, 7:8].astype(jnp.int32)
            eb_ref[...] = jnp.where(brow == NB, total, acc)


def _gate_finalize(x, Wg, bg):
    return pl.pallas_call(
        _gate_body,
        grid=(2, N_TOK // TB),
        in_specs=[
            pl.BlockSpec((TB, D_IN), lambda p, t: ((1 - p) * t, 0)),
            pl.BlockSpec((D_IN, E), lambda p, t: (0, 0)),
            pl.BlockSpec((1, E), lambda p, t: (0, 0)),
        ],
        out_specs=[
            pl.BlockSpec((TB, 1), lambda p, t: (t, 0)),
            pl.BlockSpec((TB, 1), lambda p, t: (t, 0)),
            pl.BlockSpec((TB, 128), lambda p, t: (t, 0)),
            pl.BlockSpec((TB, 128), lambda p, t: (t, 0)),
            pl.BlockSpec((EBR, 1), lambda p, t: (0, 0)),
        ],
        out_shape=[
            jax.ShapeDtypeStruct((N_TOK, 1), jnp.int32),
            jax.ShapeDtypeStruct((N_TOK, 1), jnp.int32),
            jax.ShapeDtypeStruct((N_TOK, 128), jnp.float32),
            jax.ShapeDtypeStruct((N_TOK, 128), jnp.float32),
            jax.ShapeDtypeStruct((EBR, 1), jnp.int32),
        ],
        scratch_shapes=[
            pltpu.VMEM((N_TOK, TOPK), jnp.int32),
            pltpu.VMEM((N_TOK, TOPK), jnp.float32),
            pltpu.VMEM((N_TOK, TOPK), jnp.int32),
            pltpu.VMEM((1, 16), jnp.float32),
        ],
    )(x, Wg, bg.reshape(1, E))


def _uv_body(u_ref, v_ref, o_ref):
    o_ref[0] = jnp.dot(u_ref[0], v_ref[0], preferred_element_type=jnp.float32)


def _uv_collapse(U, V):
    return pl.pallas_call(
        _uv_body,
        grid=(NLOW,),
        in_specs=[
            pl.BlockSpec((1, D_HID, RANK), lambda e: (e, 0, 0)),
            pl.BlockSpec((1, RANK, D_OUT), lambda e: (e, 0, 0)),
        ],
        out_specs=pl.BlockSpec((1, D_HID, D_OUT), lambda e: (e, 0, 0)),
        out_shape=jax.ShapeDtypeStruct((NLOW, D_HID, D_OUT), jnp.float32),
    )(U, V)


_SC_MESH = plsc.VectorSubcoreMesh(core_axis_name="c", subcore_axis_name="s")


@functools.partial(
    pl.kernel,
    mesh=_SC_MESH,
    out_type=[
        jax.ShapeDtypeStruct((NPB, D_IN), jnp.float32),
        jax.ShapeDtypeStruct((NPB, 128), jnp.float32),
    ],
    scratch_types=[
        pltpu.VMEM((TPW,), jnp.int32),
        pltpu.VMEM((TPW,), jnp.int32),
        pltpu.VMEM((TPW,), jnp.int32),
        pltpu.VMEM((TPW, D_IN), jnp.float32),
        pltpu.VMEM((TPW, 128), jnp.float32),
        pltpu.VMEM((TPW, 128), jnp.float32),
        pltpu.SemaphoreType.DMA,
        pltpu.SemaphoreType.DMA,
    ],
)
def _dispatch(d1_hbm, d2_hbm, vb1_hbm, vb2_hbm, x_hbm, xs_hbm, xsw_hbm,
              d1v, d2v, tokv, rows, w1v, w2v, sem0, sem1):
    w = lax.axis_index("s") * 2 + lax.axis_index("c")
    base = w * TPW
    pltpu.sync_copy(d1_hbm.at[pl.ds(base, TPW)], d1v)
    pltpu.sync_copy(d2_hbm.at[pl.ds(base, TPW)], d2v)
    pltpu.sync_copy(vb1_hbm.at[pl.ds(base, TPW)], w1v)
    pltpu.sync_copy(vb2_hbm.at[pl.ds(base, TPW)], w2v)
    lane = lax.iota(jnp.int32, 16)
    for j in range(TPW // 16):
        tokv[pl.ds(j * 16, 16)] = base + j * 16 + lane
    pltpu.async_copy(x_hbm.at[tokv], rows, sem0).wait()
    c0 = pltpu.async_copy(rows, xs_hbm.at[d1v], sem0)
    c1 = pltpu.async_copy(rows, xs_hbm.at[d2v], sem1)
    c0.wait()
    c1.wait()
    c2 = pltpu.async_copy(w1v, xsw_hbm.at[d1v], sem0)
    c3 = pltpu.async_copy(w2v, xsw_hbm.at[d2v], sem1)
    c2.wait()
    c3.wait()


def _erf(z):
    a = jnp.abs(z)
    t = 1.0 / (1.0 + 0.3275911 * a)
    p = t * (0.254829592 + t * (-0.284496736 + t * (1.421413741
            + t * (-1.453152027 + t * 1.061405429))))
    return jnp.sign(z) * (1.0 - p * jnp.exp(-a * a))


def _gelu_exact(x):
    return 0.5 * x * (1.0 + _erf(x * 0.7071067811865476))


def _ffn_body(eb_ref, xs_ref, w1_ref, b1_ref, w2_ref, b2_ref, ws_ref, ys_ref):
    @pl.when(pl.program_id(0) < eb_ref[NB])
    def _():
        h = (jnp.dot(xs_ref[...], w1_ref[0], preferred_element_type=jnp.float32)
             + b1_ref[0])
        h = _gelu_exact(h)
        y = (jnp.dot(h, w2_ref[0], preferred_element_type=jnp.float32)
             + b2_ref[0])
        ys_ref[...] = y * ws_ref[:, :1]


def _grouped_ffn(eb, xs, W1, b1, W2all, b2all, xsw):
    grid_spec = pltpu.PrefetchScalarGridSpec(
        num_scalar_prefetch=1,
        grid=(NB,),
        in_specs=[
            pl.BlockSpec((BLK, D_IN), lambda b, eb: (b, 0)),
            pl.BlockSpec((1, D_IN, D_HID), lambda b, eb: (eb[b], 0, 0)),
            pl.BlockSpec((1, 1, D_HID), lambda b, eb: (eb[b], 0, 0)),
            pl.BlockSpec((1, D_HID, D_OUT), lambda b, eb: (eb[b], 0, 0)),
            pl.BlockSpec((1, 1, D_OUT), lambda b, eb: (eb[b], 0, 0)),
            pl.BlockSpec((BLK, 128), lambda b, eb: (b, 0)),
        ],
        out_specs=pl.BlockSpec((BLK, D_OUT), lambda b, eb: (b, 0)),
    )
    return pl.pallas_call(
        _ffn_body,
        grid_spec=grid_spec,
        out_shape=jax.ShapeDtypeStruct((NPB, D_OUT), jnp.float32),
    )(eb, xs, W1, b1.reshape(E, 1, D_HID), W2all, b2all.reshape(E, 1, D_OUT), xsw)


@functools.partial(
    pl.kernel,
    mesh=_SC_MESH,
    out_type=jax.ShapeDtypeStruct((N_TOK, D_OUT), jnp.float32),
    scratch_types=[
        pltpu.VMEM((TPW,), jnp.int32),
        pltpu.VMEM((TPW,), jnp.int32),
        pltpu.VMEM((TPW, D_OUT), jnp.float32),
        pltpu.VMEM((TPW, D_OUT), jnp.float32),
        pltpu.SemaphoreType.DMA,
        pltpu.SemaphoreType.DMA,
    ],
)
def _combine(d1_hbm, d2_hbm, ys_hbm, out_hbm, d1v, d2v, b0, b1, sem0, sem1):
    w = lax.axis_index("s") * 2 + lax.axis_index("c")
    base = w * TPW
    pltpu.sync_copy(d1_hbm.at[pl.ds(base, TPW)], d1v)
    pltpu.sync_copy(d2_hbm.at[pl.ds(base, TPW)], d2v)
    c0 = pltpu.async_copy(ys_hbm.at[d1v], b0, sem0)
    c1 = pltpu.async_copy(ys_hbm.at[d2v], b1, sem1)
    c0.wait()
    c1.wait()

    def body(tl, _):
        for k in range(D_OUT // 16):
            b0[tl, pl.ds(k * 16, 16)] += b1[tl, pl.ds(k * 16, 16)]
        return 0

    lax.fori_loop(0, TPW, body, 0)
    pltpu.sync_copy(b0, out_hbm.at[pl.ds(base, TPW)])


def kernel(x, W1, b1, U, V, bl, W2, b2, Wg, bg):
    d1, d2, vb1, vb2, eb = _gate_finalize(x, Wg, bg)
    W2all = jnp.concatenate([_uv_collapse(U, V), W2], axis=0)
    b2all = jnp.concatenate([bl, b2], axis=0)

    d1f = d1.reshape(N_TOK)
    d2f = d2.reshape(N_TOK)
    xs, xsw = _dispatch(d1f, d2f, vb1, vb2, x)
    ys = _grouped_ffn(eb.reshape(EBR)[:NB + 1], xs, W1, b1, W2all, b2all, xsw)
    return _combine(d1f, d2f, ys)


# speedup vs baseline: 2.0423x; 1.0491x over previous
import functools

import jax
import jax.numpy as jnp
from jax import lax
from jax.experimental import pallas as pl
from jax.experimental.pallas import tpu as pltpu
from jax.experimental.pallas import tpu_sc as plsc

N_TOK = 2048
D_IN = 768
D_HID = 2048
D_OUT = 768
RANK = 64
E = 8
NLOW = 2
TOPK = 2
NFLAT = N_TOK * TOPK

TB = 256
BLK = 256
BSH = 8
NB = NFLAT // BLK + E
NPB = NB * BLK
EBR = 32

NW = 32
TPW = N_TOK // NW


def _gate_body(x_ref, wg_ref, bg_ref,
               d1_ref, d2_ref, vb1_ref, vb2_ref, eb_ref,
               ev_s, vv_s, r_s, carry_ref):
    p = pl.program_id(0)
    t = pl.program_id(1)
    lane16 = lax.broadcasted_iota(jnp.int32, (TB, 16), 1)

    @pl.when(p == 0)
    def _gate_pass():
        logits = (jnp.dot(x_ref[...], wg_ref[...],
                          preferred_element_type=jnp.float32)
                  + bg_ref[...])
        lane = lax.broadcasted_iota(jnp.int32, logits.shape, 1)
        m1 = jnp.max(logits, axis=1, keepdims=True)
        i1 = jnp.min(jnp.where(logits == m1, lane, E), axis=1, keepdims=True)
        l2 = jnp.where(lane == i1, -jnp.inf, logits)
        m2 = jnp.max(l2, axis=1, keepdims=True)
        i2 = jnp.min(jnp.where(l2 == m2, lane, E), axis=1, keepdims=True)
        e2 = jnp.exp(m2 - m1)
        s = 1.0 + e2
        ev_s[pl.ds(t * TB, TB), :] = jnp.concatenate([i1, i2], axis=1)
        vv_s[pl.ds(t * TB, TB), :] = jnp.concatenate([1.0 / s, e2 / s], axis=1)

        @pl.when(t == 0)
        def _():
            carry_ref[...] = jnp.zeros_like(carry_ref)

        oh1 = (lane16 == i1).astype(jnp.float32)
        oh2 = (lane16 == i2).astype(jnp.float32)
        ohb = oh1 + oh2
        row = lax.broadcasted_iota(jnp.int32, (TB, TB), 0)
        col = lax.broadcasted_iota(jnp.int32, (TB, TB), 1)
        ltri = (row > col).astype(jnp.float32)
        cum = (jnp.dot(ltri, ohb, preferred_element_type=jnp.float32)
               + carry_ref[...])
        r1 = jnp.sum(oh1 * cum, axis=1, keepdims=True)
        r2 = jnp.sum(oh2 * cum, axis=1, keepdims=True)
        r_s[pl.ds(t * TB, TB), :] = (
            jnp.concatenate([r1, r2], axis=1).astype(jnp.int32))
        carry_ref[...] += jnp.sum(ohb, axis=0, keepdims=True)

    @pl.when(p == 1)
    def _finalize_pass():
        cnt = carry_ref[...].astype(jnp.int32)
        nb = (cnt + (BLK - 1)) >> BSH
        nbf = nb.astype(jnp.float32)
        erow = lax.broadcasted_iota(jnp.int32, (16, 16), 0)
        ecol = lax.broadcasted_iota(jnp.int32, (16, 16), 1)
        ltri = (erow < ecol).astype(jnp.float32)
        cex = jnp.dot(nbf, ltri, preferred_element_type=jnp.float32)
        po = cex * float(BLK)
        cin = cex + nbf

        ev = ev_s[pl.ds(t * TB, TB), :]
        rr = r_s[pl.ds(t * TB, TB), :]
        vv = vv_s[pl.ds(t * TB, TB), :]
        i1 = ev[:, :1]
        i2 = ev[:, 1:2]
        oh1 = (lane16 == i1).astype(jnp.float32)
        oh2 = (lane16 == i2).astype(jnp.float32)
        d1_ref[...] = (jnp.sum(oh1 * po, axis=1, keepdims=True).astype(jnp.int32)
                       + rr[:, :1])
        d2_ref[...] = (jnp.sum(oh2 * po, axis=1, keepdims=True).astype(jnp.int32)
                       + rr[:, 1:2])

        z = jnp.zeros((TB, 128), jnp.float32)
        vb1_ref[...] = vv[:, :1] + z
        vb2_ref[...] = vv[:, 1:2] + z

        @pl.when(t == 0)
        def _():
            bcol = lax.broadcasted_iota(jnp.int32, (EBR, 16), 0).astype(jnp.float32)
            used = (bcol >= cin).astype(jnp.int32)
            acc = jnp.minimum(jnp.sum(used, axis=1, keepdims=True), E - 1)
            brow = lax.broadcasted_iota(jnp.int32, (EBR, 1), 0)
            total = cin[:, 7:8].astype(jnp.int32)
            eb_ref[...] = jnp.where(brow == NB, total, acc)


def _gate_finalize(x, Wg, bg):
    return pl.pallas_call(
        _gate_body,
        grid=(2, N_TOK // TB),
        in_specs=[
            pl.BlockSpec((TB, D_IN), lambda p, t: ((1 - p) * t, 0)),
            pl.BlockSpec((D_IN, E), lambda p, t: (0, 0)),
            pl.BlockSpec((1, E), lambda p, t: (0, 0)),
        ],
        out_specs=[
            pl.BlockSpec((TB, 1), lambda p, t: (t, 0)),
            pl.BlockSpec((TB, 1), lambda p, t: (t, 0)),
            pl.BlockSpec((TB, 128), lambda p, t: (t, 0)),
            pl.BlockSpec((TB, 128), lambda p, t: (t, 0)),
            pl.BlockSpec((EBR, 1), lambda p, t: (0, 0)),
        ],
        out_shape=[
            jax.ShapeDtypeStruct((N_TOK, 1), jnp.int32),
            jax.ShapeDtypeStruct((N_TOK, 1), jnp.int32),
            jax.ShapeDtypeStruct((N_TOK, 128), jnp.float32),
            jax.ShapeDtypeStruct((N_TOK, 128), jnp.float32),
            jax.ShapeDtypeStruct((EBR, 1), jnp.int32),
        ],
        scratch_shapes=[
            pltpu.VMEM((N_TOK, TOPK), jnp.int32),
            pltpu.VMEM((N_TOK, TOPK), jnp.float32),
            pltpu.VMEM((N_TOK, TOPK), jnp.int32),
            pltpu.VMEM((1, 16), jnp.float32),
        ],
    )(x, Wg, bg.reshape(1, E))


def _uv_body(u_ref, v_ref, o_ref):
    o_ref[0] = jnp.dot(u_ref[0], v_ref[0], preferred_element_type=jnp.float32)


def _uv_collapse(U, V):
    return pl.pallas_call(
        _uv_body,
        grid=(NLOW,),
        in_specs=[
            pl.BlockSpec((1, D_HID, RANK), lambda e: (e, 0, 0)),
            pl.BlockSpec((1, RANK, D_OUT), lambda e: (e, 0, 0)),
        ],
        out_specs=pl.BlockSpec((1, D_HID, D_OUT), lambda e: (e, 0, 0)),
        out_shape=jax.ShapeDtypeStruct((NLOW, D_HID, D_OUT), jnp.float32),
    )(U, V)


_SC_MESH = plsc.VectorSubcoreMesh(core_axis_name="c", subcore_axis_name="s")


@functools.partial(
    pl.kernel,
    mesh=_SC_MESH,
    out_type=[
        jax.ShapeDtypeStruct((NPB, D_IN), jnp.float32),
        jax.ShapeDtypeStruct((NPB, 128), jnp.float32),
    ],
    scratch_types=[
        pltpu.VMEM((TPW,), jnp.int32),
        pltpu.VMEM((TPW,), jnp.int32),
        pltpu.VMEM((TPW,), jnp.int32),
        pltpu.VMEM((TPW, D_IN), jnp.float32),
        pltpu.VMEM((TPW, 128), jnp.float32),
        pltpu.VMEM((TPW, 128), jnp.float32),
        pltpu.SemaphoreType.DMA,
        pltpu.SemaphoreType.DMA,
    ],
)
def _dispatch(d1_hbm, d2_hbm, vb1_hbm, vb2_hbm, x_hbm, xs_hbm, xsw_hbm,
              d1v, d2v, tokv, rows, w1v, w2v, sem0, sem1):
    w = lax.axis_index("s") * 2 + lax.axis_index("c")
    base = w * TPW
    pltpu.sync_copy(d1_hbm.at[pl.ds(base, TPW)], d1v)
    pltpu.sync_copy(d2_hbm.at[pl.ds(base, TPW)], d2v)
    pltpu.sync_copy(vb1_hbm.at[pl.ds(base, TPW)], w1v)
    pltpu.sync_copy(vb2_hbm.at[pl.ds(base, TPW)], w2v)
    lane = lax.iota(jnp.int32, 16)
    for j in range(TPW // 16):
        tokv[pl.ds(j * 16, 16)] = base + j * 16 + lane
    pltpu.async_copy(x_hbm.at[tokv], rows, sem0).wait()
    c0 = pltpu.async_copy(rows, xs_hbm.at[d1v], sem0)
    c1 = pltpu.async_copy(rows, xs_hbm.at[d2v], sem1)
    c0.wait()
    c1.wait()
    c2 = pltpu.async_copy(w1v, xsw_hbm.at[d1v], sem0)
    c3 = pltpu.async_copy(w2v, xsw_hbm.at[d2v], sem1)
    c2.wait()
    c3.wait()


def _erf(z):
    a = jnp.abs(z)
    t = 1.0 / (1.0 + 0.3275911 * a)
    p = t * (0.254829592 + t * (-0.284496736 + t * (1.421413741
            + t * (-1.453152027 + t * 1.061405429))))
    return jnp.sign(z) * (1.0 - p * jnp.exp(-a * a))


def _gelu_exact(x):
    return 0.5 * x * (1.0 + _erf(x * 0.7071067811865476))


def _ffn_body(eb_ref, xs_ref, w1_ref, b1_ref, w2_ref, b2_ref, ws_ref, ys_ref):
    @pl.when(pl.program_id(0) < eb_ref[NB])
    def _():
        h = (jnp.dot(xs_ref[...], w1_ref[0], preferred_element_type=jnp.float32)
             + b1_ref[0])
        h = _gelu_exact(h)
        y = (jnp.dot(h, w2_ref[0], preferred_element_type=jnp.float32)
             + b2_ref[0])
        ys_ref[...] = y * ws_ref[:, :1]


def _grouped_ffn(eb, xs, W1, b1, W2all, b2all, xsw):
    grid_spec = pltpu.PrefetchScalarGridSpec(
        num_scalar_prefetch=1,
        grid=(NB,),
        in_specs=[
            pl.BlockSpec((BLK, D_IN), lambda b, eb: (b, 0)),
            pl.BlockSpec((1, D_IN, D_HID), lambda b, eb: (eb[b], 0, 0)),
            pl.BlockSpec((1, 1, D_HID), lambda b, eb: (eb[b], 0, 0)),
            pl.BlockSpec((1, D_HID, D_OUT), lambda b, eb: (eb[b], 0, 0)),
            pl.BlockSpec((1, 1, D_OUT), lambda b, eb: (eb[b], 0, 0)),
            pl.BlockSpec((BLK, 128), lambda b, eb: (b, 0)),
        ],
        out_specs=pl.BlockSpec((BLK, D_OUT), lambda b, eb: (b, 0)),
    )
    return pl.pallas_call(
        _ffn_body,
        grid_spec=grid_spec,
        out_shape=jax.ShapeDtypeStruct((NPB, D_OUT), jnp.float32),
    )(eb, xs, W1, b1.reshape(E, 1, D_HID), W2all, b2all.reshape(E, 1, D_OUT), xsw)


@functools.partial(
    pl.kernel,
    mesh=_SC_MESH,
    out_type=jax.ShapeDtypeStruct((N_TOK, D_OUT), jnp.float32),
    scratch_types=[
        pltpu.VMEM((TPW,), jnp.int32),
        pltpu.VMEM((TPW,), jnp.int32),
        pltpu.VMEM((TPW, D_OUT), jnp.float32),
        pltpu.VMEM((TPW, D_OUT), jnp.float32),
        pltpu.SemaphoreType.DMA,
        pltpu.SemaphoreType.DMA,
    ],
)
def _combine(d1_hbm, d2_hbm, ys_hbm, out_hbm, d1v, d2v, b0, b1, sem0, sem1):
    w = lax.axis_index("s") * 2 + lax.axis_index("c")
    base = w * TPW
    pltpu.sync_copy(d1_hbm.at[pl.ds(base, TPW)], d1v)
    pltpu.sync_copy(d2_hbm.at[pl.ds(base, TPW)], d2v)
    c0 = pltpu.async_copy(ys_hbm.at[d1v], b0, sem0)
    c1 = pltpu.async_copy(ys_hbm.at[d2v], b1, sem1)
    c0.wait()
    c1.wait()

    def body(tl, _):
        for k in range(D_OUT // 16):
            b0[tl, pl.ds(k * 16, 16)] += b1[tl, pl.ds(k * 16, 16)]
        return 0

    lax.fori_loop(0, TPW, body, 0)
    pltpu.sync_copy(b0, out_hbm.at[pl.ds(base, TPW)])


def kernel(x, W1, b1, U, V, bl, W2, b2, Wg, bg):
    d1, d2, vb1, vb2, eb = _gate_finalize(x, Wg, bg)
    W2all = jnp.concatenate([_uv_collapse(U, V), W2], axis=0)
    b2all = jnp.concatenate([bl, b2], axis=0)

    d1f = d1.reshape(N_TOK)
    d2f = d2.reshape(N_TOK)
    xs, xsw = _dispatch(d1f, d2f, vb1, vb2, x)
    ys = _grouped_ffn(eb.reshape(EBR)[:NB + 1], xs, W1, b1, W2all, b2all, xsw)
    return _combine(d1f, d2f, ys)
